# Initial kernel scaffold; baseline (speedup 1.0000x reference)
import functools

import jax
import jax.numpy as jnp
from jax import lax
from jax.experimental import pallas as pl
from jax.experimental.pallas import tpu as pltpu
from jax.experimental.pallas import tpu_sc as plsc

N_NODES = 10000
N_PAD = 10240
E_EDGES = 320000
CHUNK = 128
N_CHUNKS = 79
NW = 32
E_PAD = NW * N_CHUNKS * CHUNK
ROWS_PER_TILE = N_PAD // 16


@functools.lru_cache(maxsize=None)
def _make_spmm(F):
    mesh = plsc.VectorSubcoreMesh(core_axis_name="c", subcore_axis_name="s")

    @functools.partial(
        pl.kernel,
        mesh=mesh,
        out_type=jax.ShapeDtypeStruct((2 * N_PAD, F), jnp.float32),
        scratch_types=[
            pltpu.VMEM((N_CHUNKS, CHUNK), jnp.int32),
            pltpu.VMEM((N_CHUNKS, CHUNK), jnp.int32),
            pltpu.VMEM((N_CHUNKS, CHUNK), jnp.float32),
            pltpu.VMEM((CHUNK, F), jnp.float32),
            pltpu.VMEM_SHARED((N_PAD, F), jnp.float32),
        ],
    )
    def spmm(support, src_hbm, dst_hbm, w_hbm, out, src_v, dst_v, w_v,
             rows_v, acc):
        c = lax.axis_index("c")
        s = lax.axis_index("s")
        wid = s * 2 + c
        base = wid * N_CHUNKS

        pltpu.sync_copy(src_hbm.at[pl.ds(base, N_CHUNKS)], src_v)
        pltpu.sync_copy(dst_hbm.at[pl.ds(base, N_CHUNKS)], dst_v)
        pltpu.sync_copy(w_hbm.at[pl.ds(base, N_CHUNKS)], w_v)

        zf = jnp.zeros((16,), jnp.float32)

        def zero_body(i, carry):
            for k in range(F // 16):
                rows_v[i, pl.ds(k * 16, 16)] = zf
            return carry

        lax.fori_loop(0, CHUNK, zero_body, 0)
        row0 = s * ROWS_PER_TILE
        for z in range(ROWS_PER_TILE // CHUNK):
            pltpu.sync_copy(rows_v, acc.at[pl.ds(row0 + z * CHUNK, CHUNK)])
        plsc.subcore_barrier()

        def chunk_body(j, carry):
            pltpu.sync_copy(support.at[src_v.at[j]], rows_v)

            def edge_body(e, c2):
                w = w_v[j, e]
                for k in range(F // 16):
                    sl = pl.ds(k * 16, 16)
                    rows_v[e, sl] = rows_v[e, sl] * w
                return c2

            lax.fori_loop(0, CHUNK, edge_body, 0)
            pltpu.sync_copy(rows_v, acc.at[dst_v.at[j]], add=True)
            return carry

        lax.fori_loop(0, N_CHUNKS, chunk_body, 0)
        plsc.subcore_barrier()

        pltpu.sync_copy(acc.at[pl.ds(row0, ROWS_PER_TILE)],
                        out.at[pl.ds(c * N_PAD + row0, ROWS_PER_TILE)])

    return spmm


def _mm_body(x_ref, w_ref, o_ref):
    o_ref[...] = jnp.dot(x_ref[...], w_ref[...],
                         preferred_element_type=jnp.float32)


def _combine_body(p_ref, g_ref, b_ref, w_ref, o_ref):
    h = p_ref[0] + p_ref[1]
    hs = h[:N_NODES, :]
    m = jnp.mean(hs, axis=0)
    v = jnp.mean((hs - m) ** 2, axis=0)
    hn = g_ref[...] * (h - m) * lax.rsqrt(v + 1e-5) + b_ref[...]
    he = jnp.where(hn > 0, hn, jnp.expm1(hn))
    o_ref[...] = jnp.dot(he, w_ref[...], preferred_element_type=jnp.float32)


def _final_body(p_ref, o_ref):
    h = p_ref[0, :N_NODES, :] + p_ref[1, :N_NODES, :]
    mx = jnp.max(h, axis=1, keepdims=True)
    e = jnp.exp(h - mx)
    o_ref[...] = (h - mx) - jnp.log(jnp.sum(e, axis=1, keepdims=True))


def _tc(body, out_shape, *args):
    return pl.pallas_call(
        body, out_shape=jax.ShapeDtypeStruct(out_shape, jnp.float32))(*args)


def kernel(x, edge_index, edge_weight, W1, W2, W3, W4,
           g1, b1, g2, b2, g3, b3):
    dst = edge_index[0]
    src = edge_index[1]
    pad = E_PAD - E_EDGES
    src_p = jnp.concatenate(
        [src, jnp.zeros((pad,), jnp.int32)]).reshape(NW * N_CHUNKS, CHUNK)
    dst_p = jnp.concatenate(
        [dst, jnp.zeros((pad,), jnp.int32)]).reshape(NW * N_CHUNKS, CHUNK)
    w_p = jnp.concatenate(
        [edge_weight, jnp.zeros((pad,), jnp.float32)]).reshape(
            NW * N_CHUNKS, CHUNK)
    x_pad = jnp.pad(x, ((0, N_PAD - N_NODES), (0, 0)))

    s1 = _tc(_mm_body, (N_PAD, 64), x_pad, W1)
    p1 = _make_spmm(64)(s1, src_p, dst_p, w_p).reshape(2, N_PAD, 64)
    s2 = _tc(_combine_body, (N_PAD, 32), p1,
             g1.reshape(1, -1), b1.reshape(1, -1), W2)
    p2 = _make_spmm(32)(s2, src_p, dst_p, w_p).reshape(2, N_PAD, 32)
    s3 = _tc(_combine_body, (N_PAD, 16), p2,
             g2.reshape(1, -1), b2.reshape(1, -1), W3)
    p3 = _make_spmm(16)(s3, src_p, dst_p, w_p).reshape(2, N_PAD, 16)
    s4 = _tc(_combine_body, (N_PAD, 16), p3,
             g3.reshape(1, -1), b3.reshape(1, -1), W4)
    p4 = _make_spmm(16)(s4, src_p, dst_p, w_p).reshape(2, N_PAD, 16)
    return _tc(_final_body, (N_NODES, 16), p4)


# speedup vs baseline: 7.1270x; 7.1270x over previous
import functools

import jax
import jax.numpy as jnp
from jax import lax
from jax.experimental import pallas as pl
from jax.experimental.pallas import tpu as pltpu
from jax.experimental.pallas import tpu_sc as plsc

N_NODES = 10000
N_PAD = 10240
E_EDGES = 320000
CHUNK = 128
N_CHUNKS = 80
NW = 32
E_PAD = NW * N_CHUNKS * CHUNK
ROWS_PER_TILE = N_PAD // 16


@functools.lru_cache(maxsize=None)
def _make_spmm(F):
    mesh = plsc.VectorSubcoreMesh(core_axis_name="c", subcore_axis_name="s")

    @functools.partial(
        pl.kernel,
        mesh=mesh,
        compiler_params=pltpu.CompilerParams(use_tc_tiling_on_sc=False),
        out_type=jax.ShapeDtypeStruct((2 * N_PAD, F), jnp.float32),
        scratch_types=[
            pltpu.VMEM((N_CHUNKS, CHUNK), jnp.int32),
            pltpu.VMEM((N_CHUNKS, CHUNK), jnp.int32),
            pltpu.VMEM((N_CHUNKS, CHUNK), jnp.float32),
            pltpu.VMEM((CHUNK, F), jnp.float32),
            pltpu.VMEM_SHARED((N_PAD, F), jnp.float32),
        ],
    )
    def spmm(support, src_hbm, dst_hbm, w_hbm, out, src_v, dst_v, w_v,
             rows_v, acc):
        c = lax.axis_index("c")
        s = lax.axis_index("s")
        wid = s * 2 + c
        base = wid * N_CHUNKS

        pltpu.sync_copy(src_hbm.at[pl.ds(base, N_CHUNKS)], src_v)
        pltpu.sync_copy(dst_hbm.at[pl.ds(base, N_CHUNKS)], dst_v)
        pltpu.sync_copy(w_hbm.at[pl.ds(base, N_CHUNKS)], w_v)

        zf = jnp.zeros((16,), jnp.float32)

        def zero_body(i, carry):
            for k in range(F // 16):
                rows_v[i, pl.ds(k * 16, 16)] = zf
            return carry

        lax.fori_loop(0, CHUNK, zero_body, 0)
        row0 = s * ROWS_PER_TILE
        for z in range(ROWS_PER_TILE // CHUNK):
            pltpu.sync_copy(rows_v, acc.at[pl.ds(row0 + z * CHUNK, CHUNK)])
        plsc.subcore_barrier()

        def chunk_body(j, carry):
            pltpu.sync_copy(support.at[src_v.at[j]], rows_v)

            def group_body(t, c2):
                wv = w_v[j, pl.ds(t * 16, 16)]
                for l in range(16):
                    wl = wv[l]
                    e = t * 16 + l
                    for k in range(F // 16):
                        sl = pl.ds(k * 16, 16)
                        rows_v[e, sl] = rows_v[e, sl] * wl
                return c2

            lax.fori_loop(0, CHUNK // 16, group_body, 0)
            pltpu.sync_copy(rows_v, acc.at[dst_v.at[j]], add=True)
            return carry

        lax.fori_loop(0, N_CHUNKS, chunk_body, 0)
        plsc.subcore_barrier()

        pltpu.sync_copy(acc.at[pl.ds(row0, ROWS_PER_TILE)],
                        out.at[pl.ds(c * N_PAD + row0, ROWS_PER_TILE)])

    return spmm


def _mm_body(x_ref, w_ref, o_ref):
    o_ref[...] = jnp.dot(x_ref[...], w_ref[...],
                         preferred_element_type=jnp.float32)


def _combine_body(p_ref, g_ref, b_ref, w_ref, o_ref):
    h = p_ref[0] + p_ref[1]
    hs = h[:N_NODES, :]
    m = jnp.mean(hs, axis=0)
    v = jnp.mean((hs - m) ** 2, axis=0)
    hn = g_ref[...] * (h - m) * lax.rsqrt(v + 1e-5) + b_ref[...]
    he = jnp.where(hn > 0, hn, jnp.exp(jnp.minimum(hn, 0.0)) - 1.0)
    o_ref[...] = jnp.dot(he, w_ref[...], preferred_element_type=jnp.float32)


def _final_body(p_ref, o_ref):
    h = p_ref[0, :N_NODES, :] + p_ref[1, :N_NODES, :]
    mx = jnp.max(h, axis=1, keepdims=True)
    e = jnp.exp(h - mx)
    o_ref[...] = (h - mx) - jnp.log(jnp.sum(e, axis=1, keepdims=True))


def _tc(body, out_shape, *args):
    return pl.pallas_call(
        body, out_shape=jax.ShapeDtypeStruct(out_shape, jnp.float32))(*args)


def kernel(x, edge_index, edge_weight, W1, W2, W3, W4,
           g1, b1, g2, b2, g3, b3):
    dst = edge_index[0]
    src = edge_index[1]
    pad = E_PAD - E_EDGES
    src_p = jnp.concatenate(
        [src, jnp.zeros((pad,), jnp.int32)]).reshape(NW * N_CHUNKS, CHUNK)
    dst_p = jnp.concatenate(
        [dst, jnp.zeros((pad,), jnp.int32)]).reshape(NW * N_CHUNKS, CHUNK)
    w_p = jnp.concatenate(
        [edge_weight, jnp.zeros((pad,), jnp.float32)]).reshape(
            NW * N_CHUNKS, CHUNK)
    x_pad = jnp.pad(x, ((0, N_PAD - N_NODES), (0, 0)))

    s1 = _tc(_mm_body, (N_PAD, 64), x_pad, W1)
    p1 = _make_spmm(64)(s1, src_p, dst_p, w_p).reshape(2, N_PAD, 64)
    s2 = _tc(_combine_body, (N_PAD, 32), p1,
             g1.reshape(1, -1), b1.reshape(1, -1), W2)
    p2 = _make_spmm(32)(s2, src_p, dst_p, w_p).reshape(2, N_PAD, 32)
    s3 = _tc(_combine_body, (N_PAD, 16), p2,
             g2.reshape(1, -1), b2.reshape(1, -1), W3)
    p3 = _make_spmm(16)(s3, src_p, dst_p, w_p).reshape(2, N_PAD, 16)
    s4 = _tc(_combine_body, (N_PAD, 16), p3,
             g3.reshape(1, -1), b3.reshape(1, -1), W4)
    p4 = _make_spmm(16)(s4, src_p, dst_p, w_p).reshape(2, N_PAD, 16)
    return _tc(_final_body, (N_NODES, 16), p4)


# speedup vs baseline: 9.3671x; 1.3143x over previous
import functools

import jax
import jax.numpy as jnp
from jax import lax
from jax.experimental import pallas as pl
from jax.experimental.pallas import tpu as pltpu
from jax.experimental.pallas import tpu_sc as plsc

N_NODES = 10000
N_PAD = 10240
E_EDGES = 320000
CHUNK = 128
N_CHUNKS = 80
NW = 32
E_PAD = NW * N_CHUNKS * CHUNK
ROWS_PER_TILE = N_PAD // 16


@functools.lru_cache(maxsize=None)
def _make_spmm(F):
    mesh = plsc.VectorSubcoreMesh(core_axis_name="c", subcore_axis_name="s")

    @functools.partial(
        pl.kernel,
        mesh=mesh,
        compiler_params=pltpu.CompilerParams(use_tc_tiling_on_sc=False),
        out_type=jax.ShapeDtypeStruct((2 * N_PAD, F), jnp.float32),
        scratch_types=[
            pltpu.VMEM((N_CHUNKS, CHUNK), jnp.int32),
            pltpu.VMEM((N_CHUNKS, CHUNK), jnp.int32),
            pltpu.VMEM((N_CHUNKS, CHUNK), jnp.float32),
            pltpu.VMEM((CHUNK, F), jnp.float32),
            pltpu.VMEM((CHUNK, F), jnp.float32),
            pltpu.VMEM_SHARED((N_PAD, F), jnp.float32),
            pltpu.SemaphoreType.DMA,
            pltpu.SemaphoreType.DMA,
            pltpu.SemaphoreType.DMA,
            pltpu.SemaphoreType.DMA,
        ],
    )
    def spmm(support, src_hbm, dst_hbm, w_hbm, out, src_v, dst_v, w_v,
             rows0, rows1, acc, gsem0, gsem1, ssem0, ssem1):
        c = lax.axis_index("c")
        s = lax.axis_index("s")
        wid = s * 2 + c
        base = wid * N_CHUNKS

        pltpu.sync_copy(src_hbm.at[pl.ds(base, N_CHUNKS)], src_v)
        pltpu.sync_copy(dst_hbm.at[pl.ds(base, N_CHUNKS)], dst_v)
        pltpu.sync_copy(w_hbm.at[pl.ds(base, N_CHUNKS)], w_v)

        zf = jnp.zeros((16,), jnp.float32)

        def zero_body(i, carry):
            for k in range(F // 16):
                rows0[i, pl.ds(k * 16, 16)] = zf
            return carry

        lax.fori_loop(0, CHUNK, zero_body, 0)
        row0 = s * ROWS_PER_TILE
        for z in range(ROWS_PER_TILE // CHUNK):
            pltpu.sync_copy(rows0, acc.at[pl.ds(row0 + z * CHUNK, CHUNK)])
        plsc.subcore_barrier()

        def scale(rows_v, j):
            def group_body(t, c2):
                wv = w_v[j, pl.ds(t * 16, 16)]
                for l in range(16):
                    wl = wv[l]
                    e = t * 16 + l
                    for k in range(F // 16):
                        sl = pl.ds(k * 16, 16)
                        rows_v[e, sl] = rows_v[e, sl] * wl
                return c2

            lax.fori_loop(0, CHUNK // 16, group_body, 0)

        pairs = N_CHUNKS // 2
        pltpu.async_copy(support.at[src_v.at[0]], rows0, gsem0)

        def pair_body(jj, carry):
            j0 = 2 * jj
            j1 = j0 + 1
            pltpu.make_async_copy(support.at[src_v.at[j0]], rows0,
                                  gsem0).wait()

            @pl.when(jj > 0)
            def _():
                pltpu.make_async_copy(rows1, acc.at[dst_v.at[j0]],
                                      ssem1).wait()

            pltpu.async_copy(support.at[src_v.at[j1]], rows1, gsem1)
            scale(rows0, j0)
            pltpu.async_copy(rows0, acc.at[dst_v.at[j0]], ssem0, add=True)

            pltpu.make_async_copy(support.at[src_v.at[j1]], rows1,
                                  gsem1).wait()

            @pl.when(jj < pairs - 1)
            def _():
                pltpu.make_async_copy(rows0, acc.at[dst_v.at[j0]],
                                      ssem0).wait()
                pltpu.async_copy(support.at[src_v.at[j0 + 2]], rows0, gsem0)

            scale(rows1, j1)
            pltpu.async_copy(rows1, acc.at[dst_v.at[j1]], ssem1, add=True)
            return carry

        lax.fori_loop(0, pairs, pair_body, 0)
        pltpu.make_async_copy(rows0, acc.at[dst_v.at[0]], ssem0).wait()
        pltpu.make_async_copy(rows1, acc.at[dst_v.at[0]], ssem1).wait()
        plsc.subcore_barrier()

        pltpu.sync_copy(acc.at[pl.ds(row0, ROWS_PER_TILE)],
                        out.at[pl.ds(c * N_PAD + row0, ROWS_PER_TILE)])

    return spmm


def _mm_body(x_ref, w_ref, o_ref):
    o_ref[...] = jnp.dot(x_ref[...], w_ref[...],
                         preferred_element_type=jnp.float32)


def _combine_body(p_ref, g_ref, b_ref, w_ref, o_ref):
    h = p_ref[0] + p_ref[1]
    hs = h[:N_NODES, :]
    m = jnp.mean(hs, axis=0)
    v = jnp.mean((hs - m) ** 2, axis=0)
    hn = g_ref[...] * (h - m) * lax.rsqrt(v + 1e-5) + b_ref[...]
    he = jnp.where(hn > 0, hn, jnp.exp(jnp.minimum(hn, 0.0)) - 1.0)
    o_ref[...] = jnp.dot(he, w_ref[...], preferred_element_type=jnp.float32)


def _final_body(p_ref, o_ref):
    h = p_ref[0, :N_NODES, :] + p_ref[1, :N_NODES, :]
    mx = jnp.max(h, axis=1, keepdims=True)
    e = jnp.exp(h - mx)
    o_ref[...] = (h - mx) - jnp.log(jnp.sum(e, axis=1, keepdims=True))


def _tc(body, out_shape, *args):
    return pl.pallas_call(
        body, out_shape=jax.ShapeDtypeStruct(out_shape, jnp.float32))(*args)


def kernel(x, edge_index, edge_weight, W1, W2, W3, W4,
           g1, b1, g2, b2, g3, b3):
    dst = edge_index[0]
    src = edge_index[1]
    pad = E_PAD - E_EDGES
    src_p = jnp.concatenate(
        [src, jnp.zeros((pad,), jnp.int32)]).reshape(NW * N_CHUNKS, CHUNK)
    dst_p = jnp.concatenate(
        [dst, jnp.zeros((pad,), jnp.int32)]).reshape(NW * N_CHUNKS, CHUNK)
    w_p = jnp.concatenate(
        [edge_weight, jnp.zeros((pad,), jnp.float32)]).reshape(
            NW * N_CHUNKS, CHUNK)
    x_pad = jnp.pad(x, ((0, N_PAD - N_NODES), (0, 0)))

    s1 = _tc(_mm_body, (N_PAD, 64), x_pad, W1)
    p1 = _make_spmm(64)(s1, src_p, dst_p, w_p).reshape(2, N_PAD, 64)
    s2 = _tc(_combine_body, (N_PAD, 32), p1,
             g1.reshape(1, -1), b1.reshape(1, -1), W2)
    p2 = _make_spmm(32)(s2, src_p, dst_p, w_p).reshape(2, N_PAD, 32)
    s3 = _tc(_combine_body, (N_PAD, 16), p2,
             g2.reshape(1, -1), b2.reshape(1, -1), W3)
    p3 = _make_spmm(16)(s3, src_p, dst_p, w_p).reshape(2, N_PAD, 16)
    s4 = _tc(_combine_body, (N_PAD, 16), p3,
             g3.reshape(1, -1), b3.reshape(1, -1), W4)
    p4 = _make_spmm(16)(s4, src_p, dst_p, w_p).reshape(2, N_PAD, 16)
    return _tc(_final_body, (N_NODES, 16), p4)


# speedup vs baseline: 10.6656x; 1.1386x over previous
import functools

import jax
import jax.numpy as jnp
from jax import lax
from jax.experimental import pallas as pl
from jax.experimental.pallas import tpu as pltpu
from jax.experimental.pallas import tpu_sc as plsc

N_NODES = 10000
N_PAD = 10240
E_EDGES = 320000
CHUNK = 128
N_CHUNKS = 80
NW = 32
E_PAD = NW * N_CHUNKS * CHUNK
E_TILE = N_CHUNKS * CHUNK
SUPER_BY_F = {64: 256, 32: 512, 16: 1024}
ROWS_PER_TILE = N_PAD // 16


@functools.lru_cache(maxsize=None)
def _make_spmm(F):
    mesh = plsc.VectorSubcoreMesh(core_axis_name="c", subcore_axis_name="s")
    SUPER = SUPER_BY_F[F]
    N_SUPER = E_TILE // SUPER

    @functools.partial(
        pl.kernel,
        mesh=mesh,
        compiler_params=pltpu.CompilerParams(use_tc_tiling_on_sc=False),
        out_type=jax.ShapeDtypeStruct((2 * N_PAD, F), jnp.float32),
        scratch_types=[
            pltpu.VMEM((E_TILE,), jnp.int32),
            pltpu.VMEM((N_CHUNKS, CHUNK), jnp.int32),
            pltpu.VMEM((E_TILE,), jnp.float32),
            pltpu.VMEM((SUPER, F), jnp.float32),
            pltpu.VMEM((SUPER, F), jnp.float32),
            pltpu.VMEM_SHARED((N_PAD, F), jnp.float32),
            pltpu.SemaphoreType.DMA,
            pltpu.SemaphoreType.DMA,
            pltpu.SemaphoreType.DMA,
            pltpu.SemaphoreType.DMA,
        ],
    )
    def spmm(support, src_hbm, dst_hbm, w_hbm, out, src_v, dst_v, w_v,
             rows0, rows1, acc, gsem0, gsem1, ssem0, ssem1):
        c = lax.axis_index("c")
        s = lax.axis_index("s")
        wid = s * 2 + c

        pltpu.sync_copy(src_hbm.at[pl.ds(wid * E_TILE, E_TILE)], src_v)
        pltpu.sync_copy(dst_hbm.at[pl.ds(wid * N_CHUNKS, N_CHUNKS)], dst_v)
        pltpu.sync_copy(w_hbm.at[pl.ds(wid * E_TILE, E_TILE)], w_v)

        zf = jnp.zeros((16,), jnp.float32)

        def zero_body(i, carry):
            for k in range(F // 16):
                rows0[i, pl.ds(k * 16, 16)] = zf
            return carry

        lax.fori_loop(0, CHUNK, zero_body, 0)
        row0 = s * ROWS_PER_TILE
        for z in range(ROWS_PER_TILE // CHUNK):
            pltpu.sync_copy(rows0.at[pl.ds(0, CHUNK)],
                            acc.at[pl.ds(row0 + z * CHUNK, CHUNK)])
        plsc.subcore_barrier()

        def gather(rows_v, j, gsem):
            return pltpu.make_async_copy(
                support.at[src_v.at[pl.ds(j * SUPER, SUPER)]], rows_v, gsem)

        def scale(rows_v, j):
            base = j * SUPER

            def group_body(t, c2):
                wv = w_v[pl.ds(base + t * 16, 16)]
                for l in range(16):
                    wl = wv[l]
                    e = t * 16 + l
                    for k in range(F // 16):
                        sl = pl.ds(k * 16, 16)
                        rows_v[e, sl] = rows_v[e, sl] * wl
                return c2

            lax.fori_loop(0, SUPER // 16, group_body, 0)

        qs = SUPER // CHUNK

        def scatter(rows_v, j, ssem):
            for q in range(qs):
                pltpu.async_copy(rows_v.at[pl.ds(q * CHUNK, CHUNK)],
                                 acc.at[dst_v.at[qs * j + q]], ssem, add=True)

        def drain(rows_v, ssem):
            for q in range(qs):
                pltpu.make_async_copy(rows_v.at[pl.ds(q * CHUNK, CHUNK)],
                                      acc.at[dst_v.at[0]], ssem).wait()

        pairs = N_SUPER // 2
        gather(rows0, 0, gsem0).start()

        def pair_body(jj, carry):
            j0 = 2 * jj
            j1 = j0 + 1
            gather(rows0, j0, gsem0).wait()

            @pl.when(jj > 0)
            def _():
                drain(rows1, ssem1)

            gather(rows1, j1, gsem1).start()
            scale(rows0, j0)
            scatter(rows0, j0, ssem0)

            gather(rows1, j1, gsem1).wait()

            @pl.when(jj < pairs - 1)
            def _():
                drain(rows0, ssem0)
                gather(rows0, j0 + 2, gsem0).start()

            scale(rows1, j1)
            scatter(rows1, j1, ssem1)
            return carry

        lax.fori_loop(0, pairs, pair_body, 0)
        drain(rows0, ssem0)
        drain(rows1, ssem1)
        plsc.subcore_barrier()

        pltpu.sync_copy(acc.at[pl.ds(row0, ROWS_PER_TILE)],
                        out.at[pl.ds(c * N_PAD + row0, ROWS_PER_TILE)])

    return spmm


def _mm_body(x_ref, w_ref, o_ref):
    o_ref[...] = jnp.dot(x_ref[...], w_ref[...],
                         preferred_element_type=jnp.float32)


def _combine_body(p_ref, g_ref, b_ref, w_ref, o_ref):
    h = p_ref[0] + p_ref[1]
    hs = h[:N_NODES, :]
    m = jnp.mean(hs, axis=0)
    v = jnp.mean((hs - m) ** 2, axis=0)
    hn = g_ref[...] * (h - m) * lax.rsqrt(v + 1e-5) + b_ref[...]
    he = jnp.where(hn > 0, hn, jnp.exp(jnp.minimum(hn, 0.0)) - 1.0)
    o_ref[...] = jnp.dot(he, w_ref[...], preferred_element_type=jnp.float32)


def _final_body(p_ref, o_ref):
    h = p_ref[0, :N_NODES, :] + p_ref[1, :N_NODES, :]
    mx = jnp.max(h, axis=1, keepdims=True)
    e = jnp.exp(h - mx)
    o_ref[...] = (h - mx) - jnp.log(jnp.sum(e, axis=1, keepdims=True))


def _tc(body, out_shape, *args):
    return pl.pallas_call(
        body, out_shape=jax.ShapeDtypeStruct(out_shape, jnp.float32))(*args)


def kernel(x, edge_index, edge_weight, W1, W2, W3, W4,
           g1, b1, g2, b2, g3, b3):
    dst = edge_index[0]
    src = edge_index[1]
    pad = E_PAD - E_EDGES
    src_p = jnp.concatenate([src, jnp.zeros((pad,), jnp.int32)])
    dst_p = jnp.concatenate(
        [dst, jnp.zeros((pad,), jnp.int32)]).reshape(NW * N_CHUNKS, CHUNK)
    w_p = jnp.concatenate([edge_weight, jnp.zeros((pad,), jnp.float32)])
    x_pad = jnp.pad(x, ((0, N_PAD - N_NODES), (0, 0)))

    s1 = _tc(_mm_body, (N_PAD, 64), x_pad, W1)
    p1 = _make_spmm(64)(s1, src_p, dst_p, w_p).reshape(2, N_PAD, 64)
    s2 = _tc(_combine_body, (N_PAD, 32), p1,
             g1.reshape(1, -1), b1.reshape(1, -1), W2)
    p2 = _make_spmm(32)(s2, src_p, dst_p, w_p).reshape(2, N_PAD, 32)
    s3 = _tc(_combine_body, (N_PAD, 16), p2,
             g2.reshape(1, -1), b2.reshape(1, -1), W3)
    p3 = _make_spmm(16)(s3, src_p, dst_p, w_p).reshape(2, N_PAD, 16)
    s4 = _tc(_combine_body, (N_PAD, 16), p3,
             g3.reshape(1, -1), b3.reshape(1, -1), W4)
    p4 = _make_spmm(16)(s4, src_p, dst_p, w_p).reshape(2, N_PAD, 16)
    return _tc(_final_body, (N_NODES, 16), p4)
